# baseline (speedup 1.0000x reference)
import functools

import jax
import jax.numpy as jnp
from jax import lax
from jax.experimental import pallas as pl
from jax.experimental.pallas import tpu as pltpu
from jax.experimental.pallas import tpu_sc as plsc

_BATCH = 16384
_D = 32
_L = 16
_NC = 2
_NS = 16
_NW = _NC * _NS
_BPW = _BATCH // _NW


def _body(user_hbm, item_hbm, uf_hbm, if_hbm, out_hbm,
          uidx_v, iidx_v, urows_v, irows_v, out_v, sem_u, sem_i):
    wid = lax.axis_index("s") * _NC + lax.axis_index("c")
    base = wid * _BPW

    pltpu.sync_copy(user_hbm.at[pl.ds(base, _BPW)], uidx_v)
    pltpu.sync_copy(item_hbm.at[pl.ds(base, _BPW)], iidx_v)
    cu = pltpu.async_copy(uf_hbm.at[uidx_v], urows_v, sem_u)
    ci = pltpu.async_copy(if_hbm.at[iidx_v], irows_v, sem_i)
    cu.wait()
    ci.wait()

    lane = lax.iota(jnp.int32, _L)

    def group(g, carry):
        rows = g * _L + lane
        acc = jnp.zeros((_L,), jnp.float32)
        for d in range(_D):
            col = jnp.full((_L,), d, jnp.int32)
            u = plsc.load_gather(urows_v, [rows, col])
            i = plsc.load_gather(irows_v, [rows, col])
            acc = acc + u * i
        out_v[pl.ds(g * _L, _L)] = acc
        return carry

    lax.fori_loop(0, _BPW // _L, group, 0)

    pltpu.sync_copy(out_v, out_hbm.at[pl.ds(base, _BPW)])


@jax.jit
def kernel(user, item, user_factors, item_factors):
    run = functools.partial(
        pl.kernel,
        out_type=jax.ShapeDtypeStruct((_BATCH,), jnp.float32),
        mesh=plsc.VectorSubcoreMesh(
            core_axis_name="c", subcore_axis_name="s",
            num_cores=_NC, num_subcores=_NS),
        scratch_types=[
            pltpu.VMEM((_BPW,), jnp.int32),
            pltpu.VMEM((_BPW,), jnp.int32),
            pltpu.VMEM((_BPW, _D), jnp.float32),
            pltpu.VMEM((_BPW, _D), jnp.float32),
            pltpu.VMEM((_BPW,), jnp.float32),
            pltpu.SemaphoreType.DMA,
            pltpu.SemaphoreType.DMA,
        ],
        compiler_params=pltpu.CompilerParams(
            needs_layout_passes=False, use_tc_tiling_on_sc=False),
    )(_body)
    return run(user, item, user_factors, item_factors)
